# Initial kernel scaffold; baseline (speedup 1.0000x reference)
import functools

import jax
import jax.numpy as jnp
from jax import lax
from jax.experimental import pallas as pl
from jax.experimental.pallas import tpu as pltpu
from jax.experimental.pallas import tpu_sc as plsc

_NOISE = 25_000_000
_P = 102928
_N = 256
_CLIP = 40.0

_NW = 32
_C = 3232
_PTOT = _NW * _C
_W = 3248
_WBUF = 3760
_NCH = _C // 16
_NCH_LAST = (_P - 31 * _C) // 16


def _weights_body(acol, arow, bcol, brow, scol, sign_ref, coef_out, c0_out):
    a_c = -acol[...]
    a_r = -arow[...]
    b_c = -bcol[...]
    b_r = -brow[...]
    ii = lax.broadcasted_iota(jnp.int32, (_N, _N), 0)
    jj = lax.broadcasted_iota(jnp.int32, (_N, _N), 1)

    def cnt(xr, xc, tie):
        lt = (xr < xc).astype(jnp.float32)
        eq = ((xr == xc) & tie).astype(jnp.float32)
        return jnp.sum(lt + eq, axis=1, keepdims=True)

    rank_a = cnt(a_r, a_c, jj < ii) + cnt(b_r, a_c, jj < ii)
    rank_b = cnt(a_r, b_c, jj <= ii) + cnt(b_r, b_c, jj < ii)
    w = (rank_a - rank_b) * jnp.float32(1.0 / 511.0)
    coef_out[...] = w * scol[...] * jnp.float32(1.0 / (2.0 * _N))
    c0_out[...] = sign_ref[...] * scol[0:1, :]


def _clip_body(delta_in, ssq_in, delta_out):
    total = jnp.sum(ssq_in[...])
    gnorm = jnp.sqrt(total)
    factor = _CLIP / jnp.maximum(gnorm, _CLIP)
    delta_out[...] = delta_in[...] * factor


def _sc_body(noise_hbm, seeds_hbm, coef_hbm, params_hbm,
             delta_hbm, pert_hbm, ssq_hbm,
             seeds_v, coef_v, buf0, buf1, acc, pv, sqv, sem0, sem1):
    wid = lax.axis_index("s") * 2 + lax.axis_index("c")
    s_base = wid * _C
    lanes = lax.iota(jnp.int32, 16)

    pltpu.sync_copy(seeds_hbm, seeds_v)
    pltpu.sync_copy(coef_hbm, coef_v)
    pltpu.sync_copy(params_hbm.at[pl.ds(s_base, _C)], pv)

    def extract(ref, i):
        blk = (i // 16) * 16
        v = ref[pl.ds(blk, 16)]
        return jnp.sum(jnp.where(lanes == (i - blk), v, 0))

    zeros16 = jnp.zeros((16,), jnp.float32)

    @plsc.parallel_loop(0, _NCH)
    def _(j):
        acc[pl.ds(j * 16, 16)] = zeros16

    @plsc.parallel_loop(0, (_WBUF - _W) // 16)
    def _(j):
        buf0[pl.ds(_W + j * 16, 16)] = zeros16
        buf1[pl.ds(_W + j * 16, 16)] = zeros16

    def dma_params(i):
        o = extract(seeds_v, i) + s_base
        oa = jnp.minimum((o >> 3) << 3, _NOISE - _W)
        return oa, o - oa

    def start(oa, buf, sem):
        pltpu.make_async_copy(noise_hbm.at[pl.ds(oa, _W)], buf, sem).start()

    def wait(buf, sem):
        pltpu.make_async_copy(noise_hbm.at[pl.ds(0, _W)], buf, sem).wait()

    def accumulate(buf, r, c):
        ridx = lanes + r

        @plsc.parallel_loop(0, _NCH, unroll=8)
        def _(j):
            v = plsc.load_gather(buf, [ridx + j * 16])
            plsc.addupdate(acc.at[pl.ds(j * 16, 16)], c * v)

    oa0, r0_init = dma_params(0)
    start(oa0, buf0, sem0)

    def body(g, r0):
        i1 = 2 * g + 1
        oa1, r1 = dma_params(i1)
        start(oa1, buf1, sem1)

        wait(buf0, sem0)
        accumulate(buf0, r0, extract(coef_v, 2 * g))

        @pl.when(g == 0)
        def _():
            c0 = extract(coef_v, _N)
            ridx = lanes + r0

            @plsc.parallel_loop(0, _NCH, unroll=8)
            def _(j):
                v = plsc.load_gather(buf0, [ridx + j * 16])
                plsc.addupdate(pv.at[pl.ds(j * 16, 16)], c0 * v)

        inext = 2 * g + 2
        oan, rn = dma_params(inext)

        @pl.when(inext < _N)
        def _():
            start(oan, buf0, sem0)

        wait(buf1, sem1)
        accumulate(buf1, r1, extract(coef_v, i1))
        return rn

    lax.fori_loop(0, _N // 2, body, r0_init)

    lim = jnp.where(wid == _NW - 1, _NCH_LAST, _NCH)

    def sq_body(j, svec):
        v = acc[pl.ds(j * 16, 16)]
        return svec + jnp.where(j < lim, v * v, zeros16)

    sqv[...] = lax.fori_loop(0, _NCH, sq_body, zeros16)

    pltpu.sync_copy(acc, delta_hbm.at[pl.ds(s_base, _C)])
    pltpu.sync_copy(pv, pert_hbm.at[pl.ds(s_base, _C)])
    pltpu.sync_copy(sqv, ssq_hbm.at[wid])


@functools.partial(
    pl.kernel,
    out_type=(
        jax.ShapeDtypeStruct((_PTOT,), jnp.float32),
        jax.ShapeDtypeStruct((_PTOT,), jnp.float32),
        jax.ShapeDtypeStruct((_NW, 16), jnp.float32),
    ),
    mesh=plsc.VectorSubcoreMesh(core_axis_name="c", subcore_axis_name="s"),
    scratch_types=(
        pltpu.VMEM((_N + 16,), jnp.int32),
        pltpu.VMEM((_N + 16,), jnp.float32),
        pltpu.VMEM((_WBUF,), jnp.float32),
        pltpu.VMEM((_WBUF,), jnp.float32),
        pltpu.VMEM((_C,), jnp.float32),
        pltpu.VMEM((_C,), jnp.float32),
        pltpu.VMEM((16,), jnp.float32),
        pltpu.SemaphoreType.DMA,
        pltpu.SemaphoreType.DMA,
    ),
)
def _sc_call(noise_hbm, seeds_hbm, coef_hbm, params_hbm,
             delta_hbm, pert_hbm, ssq_hbm, *rest):
    _sc_body(noise_hbm, seeds_hbm, coef_hbm, params_hbm,
             delta_hbm, pert_hbm, ssq_hbm, *rest)


def kernel(noise_table, params, perturbation_seeds, returns,
           perturbation_scales, positive_perturbation):
    f32 = jnp.float32
    acol = returns[:, 0].reshape(_N, 1)
    arow = returns[:, 0].reshape(1, _N)
    bcol = returns[:, 1].reshape(_N, 1)
    brow = returns[:, 1].reshape(1, _N)
    scol = perturbation_scales.reshape(_N, 1)
    sgn = (2.0 * jnp.asarray(positive_perturbation, f32) - 1.0).reshape(1, 1)

    coef, c0 = pl.pallas_call(
        _weights_body,
        out_shape=(
            jax.ShapeDtypeStruct((_N, 1), f32),
            jax.ShapeDtypeStruct((1, 1), f32),
        ),
    )(acol, arow, bcol, brow, scol, sgn)

    coef_ext = jnp.concatenate(
        [coef.reshape(-1), c0.reshape(-1), jnp.zeros(15, f32)])
    seeds_ext = jnp.concatenate(
        [perturbation_seeds.astype(jnp.int32), jnp.zeros(16, jnp.int32)])
    params_pad = jnp.pad(params, (0, _PTOT - _P))

    delta_raw, pert_pad, ssq = _sc_call(
        noise_table, seeds_ext, coef_ext, params_pad)

    delta2d = delta_raw.reshape(_PTOT // 128, 128)
    delta = pl.pallas_call(
        _clip_body,
        out_shape=jax.ShapeDtypeStruct((_PTOT // 128, 128), f32),
    )(delta2d, ssq).reshape(-1)[:_P]

    return delta, pert_pad[:_P]


# speedup vs baseline: 23.6419x; 23.6419x over previous
import functools

import jax
import jax.numpy as jnp
from jax import lax
from jax.experimental import pallas as pl
from jax.experimental.pallas import tpu as pltpu
from jax.experimental.pallas import tpu_sc as plsc

_NOISE = 25_000_000
_P = 102928
_N = 256
_CLIP = 40.0

_NW = 32
_C = 3232
_PTOT = _NW * _C
_W = 3248
_WBUF = 3760
_NCH = _C // 16
_NCH_LAST = (_P - 31 * _C) // 16


def _weights_body(acol, arow, bcol, brow, scol, sign_ref, coef_out, c0_out):
    a_c = -acol[...]
    a_r = -arow[...]
    b_c = -bcol[...]
    b_r = -brow[...]
    ii = lax.broadcasted_iota(jnp.int32, (_N, _N), 0)
    jj = lax.broadcasted_iota(jnp.int32, (_N, _N), 1)

    def cnt(xr, xc, tie):
        lt = (xr < xc).astype(jnp.float32)
        eq = ((xr == xc) & tie).astype(jnp.float32)
        return jnp.sum(lt + eq, axis=1, keepdims=True)

    rank_a = cnt(a_r, a_c, jj < ii) + cnt(b_r, a_c, jj < ii)
    rank_b = cnt(a_r, b_c, jj <= ii) + cnt(b_r, b_c, jj < ii)
    w = (rank_a - rank_b) * jnp.float32(1.0 / 511.0)
    coef_out[...] = w * scol[...] * jnp.float32(1.0 / (2.0 * _N))
    c0_out[...] = sign_ref[...] * scol[0:1, :]


def _clip_body(delta_in, ssq_in, delta_out):
    total = jnp.sum(ssq_in[...])
    gnorm = jnp.sqrt(total)
    factor = _CLIP / jnp.maximum(gnorm, _CLIP)
    delta_out[...] = delta_in[...] * factor


def _sc_body(noise_hbm, seeds_hbm, coef_hbm, params_hbm,
             delta_hbm, pert_hbm, ssq_hbm,
             seeds_v, coef_v, buf0, buf1, acc, pv, sqv, sem0, sem1):
    wid = lax.axis_index("s") * 2 + lax.axis_index("c")
    s_base = pl.multiple_of(wid * _C, 8)
    lanes = lax.iota(jnp.int32, 16)

    pltpu.sync_copy(seeds_hbm, seeds_v)
    pltpu.sync_copy(coef_hbm, coef_v)
    pltpu.sync_copy(params_hbm.at[pl.ds(s_base, _C)], pv)

    def extract(ref, i):
        return ref[pl.ds(i, 16)][0]

    zeros16 = jnp.zeros((16,), jnp.float32)

    @plsc.parallel_loop(0, _NCH)
    def _(j):
        acc[pl.ds(j * 16, 16)] = zeros16

    @plsc.parallel_loop(0, (_WBUF - _W) // 16)
    def _(j):
        buf0[pl.ds(_W + j * 16, 16)] = zeros16
        buf1[pl.ds(_W + j * 16, 16)] = zeros16

    def dma_params(i):
        o = extract(seeds_v, i) + s_base
        oa = pl.multiple_of(jnp.minimum((o >> 3) << 3, _NOISE - _W), 8)
        return oa, o - oa

    def start(oa, buf, sem):
        pltpu.make_async_copy(
            noise_hbm.at[pl.ds(oa, _W)], buf.at[pl.ds(0, _W)], sem).start()

    def wait(buf, sem):
        pltpu.make_async_copy(
            noise_hbm.at[pl.ds(0, _W)], buf.at[pl.ds(0, _W)], sem).wait()

    def accumulate(buf, r, c):
        ridx = lanes + r

        @plsc.parallel_loop(0, _NCH, unroll=8)
        def _(j):
            v = plsc.load_gather(buf, [ridx + j * 16])
            plsc.addupdate(acc.at[pl.ds(j * 16, 16)], c * v)

    oa0, r0_init = dma_params(0)
    start(oa0, buf0, sem0)

    def body(g, r0):
        i1 = 2 * g + 1
        oa1, r1 = dma_params(i1)
        start(oa1, buf1, sem1)

        wait(buf0, sem0)
        accumulate(buf0, r0, extract(coef_v, 2 * g))

        @pl.when(g == 0)
        def _():
            c0 = extract(coef_v, _N)
            ridx = lanes + r0

            @plsc.parallel_loop(0, _NCH, unroll=8)
            def _(j):
                v = plsc.load_gather(buf0, [ridx + j * 16])
                plsc.addupdate(pv.at[pl.ds(j * 16, 16)], c0 * v)

        inext = 2 * g + 2
        oan, rn = dma_params(inext)

        @pl.when(inext < _N)
        def _():
            start(oan, buf0, sem0)

        wait(buf1, sem1)
        accumulate(buf1, r1, extract(coef_v, i1))
        return rn

    lax.fori_loop(0, _N // 2, body, r0_init)

    lim = jnp.where(wid == _NW - 1, _NCH_LAST, _NCH)

    def sq_body(j, svec):
        v = acc[pl.ds(j * 16, 16)]
        return svec + jnp.where(j < lim, v * v, zeros16)

    sqv[...] = lax.fori_loop(0, _NCH, sq_body, zeros16)

    pltpu.sync_copy(acc, delta_hbm.at[pl.ds(s_base, _C)])
    pltpu.sync_copy(pv, pert_hbm.at[pl.ds(s_base, _C)])
    pltpu.sync_copy(sqv, ssq_hbm.at[wid])


@functools.partial(
    pl.kernel,
    out_type=(
        jax.ShapeDtypeStruct((_PTOT,), jnp.float32),
        jax.ShapeDtypeStruct((_PTOT,), jnp.float32),
        jax.ShapeDtypeStruct((_NW, 16), jnp.float32),
    ),
    mesh=plsc.VectorSubcoreMesh(core_axis_name="c", subcore_axis_name="s"),
    scratch_types=(
        pltpu.VMEM((_N + 16,), jnp.int32),
        pltpu.VMEM((_N + 16,), jnp.float32),
        pltpu.VMEM((_WBUF,), jnp.float32),
        pltpu.VMEM((_WBUF,), jnp.float32),
        pltpu.VMEM((_C,), jnp.float32),
        pltpu.VMEM((_C,), jnp.float32),
        pltpu.VMEM((16,), jnp.float32),
        pltpu.SemaphoreType.DMA,
        pltpu.SemaphoreType.DMA,
    ),
    compiler_params=pltpu.CompilerParams(needs_layout_passes=False),
)
def _sc_call(noise_hbm, seeds_hbm, coef_hbm, params_hbm,
             delta_hbm, pert_hbm, ssq_hbm, *rest):
    _sc_body(noise_hbm, seeds_hbm, coef_hbm, params_hbm,
             delta_hbm, pert_hbm, ssq_hbm, *rest)


def kernel(noise_table, params, perturbation_seeds, returns,
           perturbation_scales, positive_perturbation):
    f32 = jnp.float32
    acol = returns[:, 0].reshape(_N, 1)
    arow = returns[:, 0].reshape(1, _N)
    bcol = returns[:, 1].reshape(_N, 1)
    brow = returns[:, 1].reshape(1, _N)
    scol = perturbation_scales.reshape(_N, 1)
    sgn = (2.0 * jnp.asarray(positive_perturbation, f32) - 1.0).reshape(1, 1)

    coef, c0 = pl.pallas_call(
        _weights_body,
        out_shape=(
            jax.ShapeDtypeStruct((_N, 1), f32),
            jax.ShapeDtypeStruct((1, 1), f32),
        ),
    )(acol, arow, bcol, brow, scol, sgn)

    coef_ext = jnp.concatenate(
        [coef.reshape(-1), c0.reshape(-1), jnp.zeros(15, f32)])
    seeds_ext = jnp.concatenate(
        [perturbation_seeds.astype(jnp.int32), jnp.zeros(16, jnp.int32)])
    params_pad = jnp.pad(params, (0, _PTOT - _P))

    delta_raw, pert_pad, ssq = _sc_call(
        noise_table, seeds_ext, coef_ext, params_pad)

    delta2d = delta_raw.reshape(_PTOT // 128, 128)
    delta = pl.pallas_call(
        _clip_body,
        out_shape=jax.ShapeDtypeStruct((_PTOT // 128, 128), f32),
    )(delta2d, ssq).reshape(-1)[:_P]

    return delta, pert_pad[:_P]


# speedup vs baseline: 23.6796x; 1.0016x over previous
import functools

import jax
import jax.numpy as jnp
from jax import lax
from jax.experimental import pallas as pl
from jax.experimental.pallas import tpu as pltpu
from jax.experimental.pallas import tpu_sc as plsc

_NOISE = 25_000_000
_P = 102928
_N = 256
_CLIP = 40.0

_NW = 32
_C = 3232
_PTOT = _NW * _C
_W = 3248
_WBUF = 3760
_NCH = _C // 16
_NCH_LAST = (_P - 31 * _C) // 16


def _weights_body(acol, arow, bcol, brow, scol, sign_ref, coef_out, c0_out):
    a_c = -acol[...]
    a_r = -arow[...]
    b_c = -bcol[...]
    b_r = -brow[...]
    ii = lax.broadcasted_iota(jnp.int32, (_N, _N), 0)
    jj = lax.broadcasted_iota(jnp.int32, (_N, _N), 1)

    def cnt(xr, xc, tie):
        lt = (xr < xc).astype(jnp.float32)
        eq = ((xr == xc) & tie).astype(jnp.float32)
        return jnp.sum(lt + eq, axis=1, keepdims=True)

    rank_a = cnt(a_r, a_c, jj < ii) + cnt(b_r, a_c, jj < ii)
    rank_b = cnt(a_r, b_c, jj <= ii) + cnt(b_r, b_c, jj < ii)
    w = (rank_a - rank_b) * jnp.float32(1.0 / 511.0)
    coef_out[...] = w * scol[...] * jnp.float32(1.0 / (2.0 * _N))
    c0_out[...] = sign_ref[...] * scol[0:1, :]


def _clip_body(delta_in, ssq_in, delta_out):
    total = jnp.sum(ssq_in[...])
    gnorm = jnp.sqrt(total)
    factor = _CLIP / jnp.maximum(gnorm, _CLIP)
    delta_out[...] = delta_in[...] * factor


def _sc_body(noise_hbm, seeds_hbm, coef_hbm, params_hbm,
             delta_hbm, pert_hbm, ssq_hbm,
             seeds_v, coef_v, buf0, buf1, acc, pv, sqv, sem0, sem1):
    wid = lax.axis_index("s") * 2 + lax.axis_index("c")
    s_base = pl.multiple_of(wid * _C, 8)
    lanes = lax.iota(jnp.int32, 16)

    pltpu.sync_copy(seeds_hbm, seeds_v)
    pltpu.sync_copy(coef_hbm, coef_v)
    pltpu.sync_copy(params_hbm.at[pl.ds(s_base, _C)], pv)

    def extract(ref, i):
        return ref[pl.ds(i, 16)][0]

    zeros16 = jnp.zeros((16,), jnp.float32)

    @plsc.parallel_loop(0, _NCH)
    def _(j):
        acc[pl.ds(j * 16, 16)] = zeros16

    @plsc.parallel_loop(0, (_WBUF - _W) // 16)
    def _(j):
        buf0[pl.ds(_W + j * 16, 16)] = zeros16
        buf1[pl.ds(_W + j * 16, 16)] = zeros16

    def dma_params(i):
        o = extract(seeds_v, i) + s_base
        oa = pl.multiple_of(jnp.minimum((o >> 3) << 3, _NOISE - _W), 8)
        return oa, o - oa

    def start(oa, buf, sem):
        pltpu.make_async_copy(
            noise_hbm.at[pl.ds(oa, _W)], buf.at[pl.ds(0, _W)], sem).start()

    def wait(buf, sem):
        pltpu.make_async_copy(
            noise_hbm.at[pl.ds(0, _W)], buf.at[pl.ds(0, _W)], sem).wait()

    def accumulate(buf, r, c):
        @plsc.parallel_loop(0, _NCH, unroll=8)
        def _(j):
            v = buf[pl.ds(r + j * 16, 16)]
            plsc.addupdate(acc.at[pl.ds(j * 16, 16)], c * v)

    oa0, r0_init = dma_params(0)
    start(oa0, buf0, sem0)

    def body(g, r0):
        i1 = 2 * g + 1
        oa1, r1 = dma_params(i1)
        start(oa1, buf1, sem1)

        wait(buf0, sem0)
        accumulate(buf0, r0, extract(coef_v, 2 * g))

        @pl.when(g == 0)
        def _():
            c0 = extract(coef_v, _N)

            @plsc.parallel_loop(0, _NCH, unroll=8)
            def _(j):
                v = buf0[pl.ds(r0 + j * 16, 16)]
                plsc.addupdate(pv.at[pl.ds(j * 16, 16)], c0 * v)

        inext = 2 * g + 2
        oan, rn = dma_params(inext)

        @pl.when(inext < _N)
        def _():
            start(oan, buf0, sem0)

        wait(buf1, sem1)
        accumulate(buf1, r1, extract(coef_v, i1))
        return rn

    lax.fori_loop(0, _N // 2, body, r0_init)

    lim = jnp.where(wid == _NW - 1, _NCH_LAST, _NCH)

    def sq_body(j, svec):
        v = acc[pl.ds(j * 16, 16)]
        return svec + jnp.where(j < lim, v * v, zeros16)

    sqv[...] = lax.fori_loop(0, _NCH, sq_body, zeros16)

    pltpu.sync_copy(acc, delta_hbm.at[pl.ds(s_base, _C)])
    pltpu.sync_copy(pv, pert_hbm.at[pl.ds(s_base, _C)])
    pltpu.sync_copy(sqv, ssq_hbm.at[wid])


@functools.partial(
    pl.kernel,
    out_type=(
        jax.ShapeDtypeStruct((_PTOT,), jnp.float32),
        jax.ShapeDtypeStruct((_PTOT,), jnp.float32),
        jax.ShapeDtypeStruct((_NW, 16), jnp.float32),
    ),
    mesh=plsc.VectorSubcoreMesh(core_axis_name="c", subcore_axis_name="s"),
    scratch_types=(
        pltpu.VMEM((_N + 16,), jnp.int32),
        pltpu.VMEM((_N + 16,), jnp.float32),
        pltpu.VMEM((_WBUF,), jnp.float32),
        pltpu.VMEM((_WBUF,), jnp.float32),
        pltpu.VMEM((_C,), jnp.float32),
        pltpu.VMEM((_C,), jnp.float32),
        pltpu.VMEM((16,), jnp.float32),
        pltpu.SemaphoreType.DMA,
        pltpu.SemaphoreType.DMA,
    ),
    compiler_params=pltpu.CompilerParams(needs_layout_passes=False),
)
def _sc_call(noise_hbm, seeds_hbm, coef_hbm, params_hbm,
             delta_hbm, pert_hbm, ssq_hbm, *rest):
    _sc_body(noise_hbm, seeds_hbm, coef_hbm, params_hbm,
             delta_hbm, pert_hbm, ssq_hbm, *rest)


def kernel(noise_table, params, perturbation_seeds, returns,
           perturbation_scales, positive_perturbation):
    f32 = jnp.float32
    acol = returns[:, 0].reshape(_N, 1)
    arow = returns[:, 0].reshape(1, _N)
    bcol = returns[:, 1].reshape(_N, 1)
    brow = returns[:, 1].reshape(1, _N)
    scol = perturbation_scales.reshape(_N, 1)
    sgn = (2.0 * jnp.asarray(positive_perturbation, f32) - 1.0).reshape(1, 1)

    coef, c0 = pl.pallas_call(
        _weights_body,
        out_shape=(
            jax.ShapeDtypeStruct((_N, 1), f32),
            jax.ShapeDtypeStruct((1, 1), f32),
        ),
    )(acol, arow, bcol, brow, scol, sgn)

    coef_ext = jnp.concatenate(
        [coef.reshape(-1), c0.reshape(-1), jnp.zeros(15, f32)])
    seeds_ext = jnp.concatenate(
        [perturbation_seeds.astype(jnp.int32), jnp.zeros(16, jnp.int32)])
    params_pad = jnp.pad(params, (0, _PTOT - _P))

    delta_raw, pert_pad, ssq = _sc_call(
        noise_table, seeds_ext, coef_ext, params_pad)

    delta2d = delta_raw.reshape(_PTOT // 128, 128)
    delta = pl.pallas_call(
        _clip_body,
        out_shape=jax.ShapeDtypeStruct((_PTOT // 128, 128), f32),
    )(delta2d, ssq).reshape(-1)[:_P]

    return delta, pert_pad[:_P]


# speedup vs baseline: 37.1687x; 1.5696x over previous
import functools

import jax
import jax.numpy as jnp
from jax import lax
from jax.experimental import pallas as pl
from jax.experimental.pallas import tpu as pltpu
from jax.experimental.pallas import tpu_sc as plsc

_NOISE = 25_000_000
_P = 102928
_N = 256
_CLIP = 40.0

_NW = 32
_C = 3232
_PTOT = _NW * _C
_W = 3248
_WBUF = 3760
_NCH = _C // 16
_NCH_LAST = (_P - 31 * _C) // 16


def _weights_body(acol, arow, bcol, brow, scol, sign_ref, coef_out, c0_out):
    a_c = -acol[...]
    a_r = -arow[...]
    b_c = -bcol[...]
    b_r = -brow[...]
    ii = lax.broadcasted_iota(jnp.int32, (_N, _N), 0)
    jj = lax.broadcasted_iota(jnp.int32, (_N, _N), 1)

    def cnt(xr, xc, tie):
        lt = (xr < xc).astype(jnp.float32)
        eq = ((xr == xc) & tie).astype(jnp.float32)
        return jnp.sum(lt + eq, axis=1, keepdims=True)

    rank_a = cnt(a_r, a_c, jj < ii) + cnt(b_r, a_c, jj < ii)
    rank_b = cnt(a_r, b_c, jj <= ii) + cnt(b_r, b_c, jj < ii)
    w = (rank_a - rank_b) * jnp.float32(1.0 / 511.0)
    coef_out[...] = w * scol[...] * jnp.float32(1.0 / (2.0 * _N))
    c0_out[...] = sign_ref[...] * scol[0:1, :]


def _clip_body(delta_in, ssq_in, delta_out):
    total = jnp.sum(ssq_in[...])
    gnorm = jnp.sqrt(total)
    factor = _CLIP / jnp.maximum(gnorm, _CLIP)
    delta_out[...] = delta_in[...] * factor


_K = 8


def _sc_body(noise_hbm, seeds_hbm, coef_hbm, params_hbm,
             delta_hbm, pert_hbm, ssq_hbm,
             seeds_v, coef_v, acc, pv, sqv, *rest):
    bufs = rest[:_K]
    sems = rest[_K:]
    wid = lax.axis_index("s") * 2 + lax.axis_index("c")
    s_base = pl.multiple_of(wid * _C, 8)
    lanes = lax.iota(jnp.int32, 16)

    pltpu.sync_copy(seeds_hbm, seeds_v)
    pltpu.sync_copy(coef_hbm, coef_v)
    pltpu.sync_copy(params_hbm.at[pl.ds(s_base, _C)], pv)

    def extract(ref, i):
        return ref[pl.ds(i, 16)][0]

    zeros16 = jnp.zeros((16,), jnp.float32)

    @plsc.parallel_loop(0, _NCH)
    def _(j):
        acc[pl.ds(j * 16, 16)] = zeros16

    @plsc.parallel_loop(0, (_WBUF - _W) // 16)
    def _(j):
        for _b in range(_K):
            bufs[_b][pl.ds(_W + j * 16, 16)] = zeros16

    def dma_params(i):
        o = extract(seeds_v, i) + s_base
        oa = pl.multiple_of(jnp.minimum((o >> 3) << 3, _NOISE - _W), 8)
        return oa, o - oa

    def start(oa, buf, sem):
        pltpu.make_async_copy(
            noise_hbm.at[pl.ds(oa, _W)], buf.at[pl.ds(0, _W)], sem).start()

    def wait(buf, sem):
        pltpu.make_async_copy(
            noise_hbm.at[pl.ds(0, _W)], buf.at[pl.ds(0, _W)], sem).wait()

    def accumulate(buf, r, c):
        @plsc.parallel_loop(0, _NCH, unroll=8)
        def _(j):
            v = buf[pl.ds(r + j * 16, 16)]
            plsc.addupdate(acc.at[pl.ds(j * 16, 16)], c * v)

    rs0 = []
    for b in range(_K):
        oa_b, r_b = dma_params(b)
        start(oa_b, bufs[b], sems[b])
        rs0.append(r_b)

    def body(g, rs):
        rs = list(rs)
        for b in range(_K):
            i = g * _K + b
            wait(bufs[b], sems[b])
            accumulate(bufs[b], rs[b], extract(coef_v, i))

            if b == 0:
                @pl.when(g == 0)
                def _():
                    c0 = extract(coef_v, _N)
                    r0 = rs[0]

                    @plsc.parallel_loop(0, _NCH, unroll=8)
                    def _(j):
                        v = bufs[0][pl.ds(r0 + j * 16, 16)]
                        plsc.addupdate(pv.at[pl.ds(j * 16, 16)], c0 * v)

            inext = i + _K
            oan, rn = dma_params(inext)

            @pl.when(inext < _N)
            def _():
                start(oan, bufs[b], sems[b])

            rs[b] = rn
        return tuple(rs)

    lax.fori_loop(0, _N // _K, body, tuple(rs0))

    lim = jnp.where(wid == _NW - 1, _NCH_LAST, _NCH)

    def sq_body(j, svec):
        v = acc[pl.ds(j * 16, 16)]
        return svec + jnp.where(j < lim, v * v, zeros16)

    sqv[...] = lax.fori_loop(0, _NCH, sq_body, zeros16)

    pltpu.sync_copy(acc, delta_hbm.at[pl.ds(s_base, _C)])
    pltpu.sync_copy(pv, pert_hbm.at[pl.ds(s_base, _C)])
    pltpu.sync_copy(sqv, ssq_hbm.at[wid])


@functools.partial(
    pl.kernel,
    out_type=(
        jax.ShapeDtypeStruct((_PTOT,), jnp.float32),
        jax.ShapeDtypeStruct((_PTOT,), jnp.float32),
        jax.ShapeDtypeStruct((_NW, 16), jnp.float32),
    ),
    mesh=plsc.VectorSubcoreMesh(core_axis_name="c", subcore_axis_name="s"),
    scratch_types=(
        pltpu.VMEM((_N + 32,), jnp.int32),
        pltpu.VMEM((_N + 32,), jnp.float32),
        pltpu.VMEM((_C,), jnp.float32),
        pltpu.VMEM((_C,), jnp.float32),
        pltpu.VMEM((16,), jnp.float32),
    ) + tuple(pltpu.VMEM((_WBUF,), jnp.float32) for _ in range(_K))
      + tuple(pltpu.SemaphoreType.DMA for _ in range(_K)),
    compiler_params=pltpu.CompilerParams(needs_layout_passes=False),
)
def _sc_call(noise_hbm, seeds_hbm, coef_hbm, params_hbm,
             delta_hbm, pert_hbm, ssq_hbm, *rest):
    _sc_body(noise_hbm, seeds_hbm, coef_hbm, params_hbm,
             delta_hbm, pert_hbm, ssq_hbm, *rest)


def kernel(noise_table, params, perturbation_seeds, returns,
           perturbation_scales, positive_perturbation):
    f32 = jnp.float32
    acol = returns[:, 0].reshape(_N, 1)
    arow = returns[:, 0].reshape(1, _N)
    bcol = returns[:, 1].reshape(_N, 1)
    brow = returns[:, 1].reshape(1, _N)
    scol = perturbation_scales.reshape(_N, 1)
    sgn = (2.0 * jnp.asarray(positive_perturbation, f32) - 1.0).reshape(1, 1)

    coef, c0 = pl.pallas_call(
        _weights_body,
        out_shape=(
            jax.ShapeDtypeStruct((_N, 1), f32),
            jax.ShapeDtypeStruct((1, 1), f32),
        ),
    )(acol, arow, bcol, brow, scol, sgn)

    coef_ext = jnp.concatenate(
        [coef.reshape(-1), c0.reshape(-1), jnp.zeros(31, f32)])
    seeds_ext = jnp.concatenate(
        [perturbation_seeds.astype(jnp.int32), jnp.zeros(32, jnp.int32)])
    params_pad = jnp.pad(params, (0, _PTOT - _P))

    delta_raw, pert_pad, ssq = _sc_call(
        noise_table, seeds_ext, coef_ext, params_pad)

    delta2d = delta_raw.reshape(_PTOT // 128, 128)
    delta = pl.pallas_call(
        _clip_body,
        out_shape=jax.ShapeDtypeStruct((_PTOT // 128, 128), f32),
    )(delta2d, ssq).reshape(-1)[:_P]

    return delta, pert_pad[:_P]
